# baseline (speedup 1.0000x reference)
import functools

import jax
import jax.numpy as jnp
from jax import lax
from jax.experimental import pallas as pl
from jax.experimental.pallas import tpu as pltpu

B = 16384
C = 1000
LANES = 128
BLK = 2048
NCHUNK = B // BLK


def _body(cols_ref, a_any, b_any, g_any, out_ref, a_v, b_v, g_v, sems):
    copies = []
    for t, (src, dst) in enumerate(((a_any, a_v), (b_any, b_v),
                                    (g_any, g_v))):
        col0 = pl.multiple_of(lax.div(cols_ref[t], LANES) * LANES, LANES)
        cp = pltpu.make_async_copy(
            src.at[:, pl.ds(col0, LANES)], dst, sems.at[t])
        cp.start()
        copies.append(cp)
    for cp in copies:
        cp.wait()

    lane_ids = lax.broadcasted_iota(jnp.int32, (BLK, LANES), 1)
    ones = jnp.ones((LANES, 1), dtype=jnp.float32)

    for i in range(NCHUNK):
        sl = pl.ds(i * BLK, BLK)

        def extract(ref, t):
            lane = lax.rem(cols_ref[t], LANES)
            masked = jnp.where(lane_ids == lane, ref[sl, :], 0.0)
            return jax.lax.dot_general(
                masked, ones, (((1,), (0,)), ((), ())),
                preferred_element_type=jnp.float32)

        a = extract(a_v, 0)
        b = extract(b_v, 1)
        g = extract(g_v, 2)
        x = jnp.maximum(1.0 - jnp.exp(g), 1e-8)
        val = a + b - jnp.log(x)
        out_ref[sl] = jnp.maximum(val, 0.0).reshape(BLK)


@jax.jit
def _transition_loss_tc(a, b, g, cols):
    return pl.pallas_call(
        _body,
        grid_spec=pltpu.PrefetchScalarGridSpec(
            num_scalar_prefetch=1,
            grid=(),
            in_specs=[pl.BlockSpec(memory_space=pl.ANY)] * 3,
            out_specs=pl.BlockSpec(memory_space=pltpu.VMEM),
            scratch_shapes=[
                pltpu.VMEM((B, LANES), jnp.float32),
                pltpu.VMEM((B, LANES), jnp.float32),
                pltpu.VMEM((B, LANES), jnp.float32),
                pltpu.SemaphoreType.DMA((3,)),
            ],
        ),
        out_shape=jax.ShapeDtypeStruct((B,), jnp.float32),
    )(cols, a, b, g)


def kernel(log_y_alpha, log_y_beta, log_y_gamma,
           alpha_index, beta_index, gamma_index):
    cols = jnp.stack([
        jnp.asarray(alpha_index, dtype=jnp.int32),
        jnp.asarray(beta_index, dtype=jnp.int32),
        jnp.asarray(gamma_index, dtype=jnp.int32),
    ])
    return _transition_loss_tc(log_y_alpha, log_y_beta, log_y_gamma, cols)


# speedup vs baseline: 34.7653x; 34.7653x over previous
import jax
import jax.numpy as jnp
from jax import lax
from jax.experimental import pallas as pl
from jax.experimental.pallas import tpu as pltpu

B = 16384
C = 1000
SUB = 8


def _body(cols_ref, a_any, b_any, g_any, out_ref, a_v, b_v, g_v, sems):
    copies = []
    for t, (src, dst) in enumerate(((a_any, a_v), (b_any, b_v),
                                    (g_any, g_v))):
        r0 = pl.multiple_of(lax.div(cols_ref[t], SUB) * SUB, SUB)
        cp = pltpu.make_async_copy(
            src.at[pl.ds(r0, SUB), :], dst, sems.at[t])
        cp.start()
        copies.append(cp)
    for cp in copies:
        cp.wait()

    sub_ids = lax.broadcasted_iota(jnp.int32, (SUB, B), 0)

    def extract(ref, t):
        row = lax.rem(cols_ref[t], SUB)
        return jnp.sum(jnp.where(sub_ids == row, ref[...], 0.0), axis=0)

    a = extract(a_v, 0)
    b = extract(b_v, 1)
    g = extract(g_v, 2)
    x = jnp.maximum(1.0 - jnp.exp(g), 1e-8)
    val = a + b - jnp.log(x)
    out_ref[...] = jnp.maximum(val, 0.0)


@jax.jit
def _transition_loss_tc(at, bt, gt, cols):
    return pl.pallas_call(
        _body,
        grid_spec=pltpu.PrefetchScalarGridSpec(
            num_scalar_prefetch=1,
            grid=(),
            in_specs=[pl.BlockSpec(memory_space=pl.ANY)] * 3,
            out_specs=pl.BlockSpec(memory_space=pltpu.VMEM),
            scratch_shapes=[
                pltpu.VMEM((SUB, B), jnp.float32),
                pltpu.VMEM((SUB, B), jnp.float32),
                pltpu.VMEM((SUB, B), jnp.float32),
                pltpu.SemaphoreType.DMA((3,)),
            ],
        ),
        out_shape=jax.ShapeDtypeStruct((B,), jnp.float32),
    )(cols, at, bt, gt)


def kernel(log_y_alpha, log_y_beta, log_y_gamma,
           alpha_index, beta_index, gamma_index):
    cols = jnp.stack([
        jnp.asarray(alpha_index, dtype=jnp.int32),
        jnp.asarray(beta_index, dtype=jnp.int32),
        jnp.asarray(gamma_index, dtype=jnp.int32),
    ])
    return _transition_loss_tc(
        log_y_alpha.T, log_y_beta.T, log_y_gamma.T, cols)
